# baseline (speedup 1.0000x reference)
import functools

import jax
import jax.numpy as jnp
from jax import lax
from jax.experimental import pallas as pl
from jax.experimental.pallas import tpu as pltpu
from jax.experimental.pallas import tpu_sc as plsc

NC, NS, L = 2, 16, 16
NW = NC * NS
ROWS, COLS = 128, 100000
RPW = ROWS // NW
CHUNK = 50000
CPR = COLS // CHUNK
NCH = RPW * CPR
VECS = CHUNK // L
UNROLL = 5
NEG = float("-inf")


def _sc_body(pred_hbm, true_hbm, out_hbm, true_v, buf0, buf1, part_v,
             sem0, sem1):
    c = lax.axis_index("c")
    s = lax.axis_index("s")
    wid = s * NC + c
    base = wid * (RPW * COLS)

    pltpu.sync_copy(true_hbm, true_v)

    bufs = (buf0, buf1)
    sems = (sem0, sem1)
    descs = [None] * NCH
    descs[0] = pltpu.async_copy(pred_hbm.at[pl.ds(base, CHUNK)], bufs[0],
                                sems[0])

    lane = lax.iota(jnp.int32, L)
    total = jnp.float32(0.0)
    for r in range(RPW):
        row_idx = jnp.broadcast_to(wid * RPW + r, (L,)).astype(jnp.int32)
        t_r = plsc.load_gather(true_v, [row_idx])
        acc = jnp.full((L,), NEG, jnp.float32)
        tl = jnp.full((L,), NEG, jnp.float32)
        for h in range(CPR):
            ch = r * CPR + h
            b = ch % 2
            if ch + 1 < NCH:
                nb = (ch + 1) % 2
                descs[ch + 1] = pltpu.async_copy(
                    pred_hbm.at[pl.ds(base + (ch + 1) * CHUNK, CHUNK)],
                    bufs[nb], sems[nb])
            descs[ch].wait()
            buf = bufs[b]

            p = t_r - h * CHUNK
            inr = (p >= 0) & (p < CHUNK)
            pc = jnp.clip(p, 0, CHUNK - 1)
            g = plsc.load_gather(buf, [pc])
            tl = jnp.where(inr, g, tl)
            plsc.store_scatter(buf, [pc], jnp.full((L,), NEG, jnp.float32),
                               mask=inr & (lane == 0))

            accs = (acc,) + tuple(
                jnp.full((L,), NEG, jnp.float32) for _ in range(UNROLL - 1))

            def scan_body(i, a, _buf=buf):
                o = i * (UNROLL * L)
                return tuple(
                    jnp.maximum(a[k], _buf[pl.ds(o + k * L, L)])
                    for k in range(UNROLL))

            accs = plsc.parallel_loop(0, VECS // UNROLL, 1,
                                      carry=accs)(scan_body)
            a = accs[0]
            for k in range(1, UNROLL):
                a = jnp.maximum(a, accs[k])
            acc = a
        target = jnp.max(acc)
        true_logit = jnp.max(tl)
        total = total + (target - true_logit)

    part_v[...] = jnp.broadcast_to(total, (L,))
    pltpu.sync_copy(part_v, out_hbm.at[wid])


_sc_kernel = functools.partial(
    pl.kernel,
    out_type=jax.ShapeDtypeStruct((NW, L), jnp.float32),
    mesh=plsc.VectorSubcoreMesh(core_axis_name="c", subcore_axis_name="s",
                                num_cores=NC, num_subcores=NS),
    compiler_params=pltpu.CompilerParams(needs_layout_passes=False),
    scratch_types=[
        pltpu.VMEM((ROWS,), jnp.int32),
        pltpu.VMEM((CHUNK,), jnp.float32),
        pltpu.VMEM((CHUNK,), jnp.float32),
        pltpu.VMEM((L,), jnp.float32),
        pltpu.SemaphoreType.DMA,
        pltpu.SemaphoreType.DMA,
    ],
)(_sc_body)


def _fin_body(x_ref, o_ref):
    o_ref[...] = jnp.sum(x_ref[:, 0:1]).reshape(1, 1)


def _finish(partials):
    return pl.pallas_call(
        _fin_body,
        out_shape=jax.ShapeDtypeStruct((1, 1), jnp.float32),
    )(partials)


@jax.jit
def kernel(pred, true):
    partials = _sc_kernel(pred.reshape(-1), true.astype(jnp.int32))
    return _finish(partials)[0, 0]
